# baseline (speedup 1.0000x reference)
import jax
import jax.numpy as jnp
from jax.experimental import pallas as pl


_S_BLK = 512


def _add_kernel(x_ref, e_ref, o_ref):
    o_ref[...] = x_ref[...] + e_ref[None]


def kernel(inputs, embeddings):
    batch, seq_len, dim = inputs.shape
    pos = embeddings[:seq_len]
    ns = seq_len // _S_BLK
    return pl.pallas_call(
        _add_kernel,
        grid=(ns,),
        in_specs=[
            pl.BlockSpec((batch, _S_BLK, dim), lambda s: (0, s, 0)),
            pl.BlockSpec((_S_BLK, dim), lambda s: (s, 0)),
        ],
        out_specs=pl.BlockSpec((batch, _S_BLK, dim), lambda s: (0, s, 0)),
        out_shape=jax.ShapeDtypeStruct(inputs.shape, inputs.dtype),
    )(inputs, pos)


# speedup vs baseline: 1.0047x; 1.0047x over previous
import jax
import jax.numpy as jnp
from jax.experimental import pallas as pl
from jax.experimental.pallas import tpu as pltpu


_S_BLK = 2048


def _add_kernel(x_ref, e_ref, o_ref):
    o_ref[...] = x_ref[...] + e_ref[...]


def kernel(inputs, embeddings):
    batch, seq_len, dim = inputs.shape
    pos = embeddings[:seq_len]
    ns = seq_len // _S_BLK
    return pl.pallas_call(
        _add_kernel,
        grid=(ns, batch),
        in_specs=[
            pl.BlockSpec((1, _S_BLK, dim), lambda s, b: (b, s, 0)),
            pl.BlockSpec((_S_BLK, dim), lambda s, b: (s, 0)),
        ],
        out_specs=pl.BlockSpec((1, _S_BLK, dim), lambda s, b: (b, s, 0)),
        out_shape=jax.ShapeDtypeStruct(inputs.shape, inputs.dtype),
        compiler_params=pltpu.CompilerParams(
            dimension_semantics=("parallel", "parallel")),
    )(inputs, pos)
